# Initial kernel scaffold; baseline (speedup 1.0000x reference)
import jax
import jax.numpy as jnp
from jax.experimental import pallas as pl


def kernel(x, edge_attrs, edge_index, W1, b1, W2, b2, Wl, bl):
    raise NotImplementedError("write your pallas kernel here")


# speedup vs baseline: 12.9037x; 12.9037x over previous
import functools

import jax
import jax.numpy as jnp
from jax import lax
from jax.experimental import pallas as pl
from jax.experimental.pallas import tpu as pltpu
from jax.experimental.pallas import tpu_sc as plsc

F32 = jnp.float32
_NS = 16
_NC = 2
_CH = 128
_BR = 400


def _mesh():
    return plsc.VectorSubcoreMesh(core_axis_name="c", subcore_axis_name="s")


def _make_deg(N, E):
    CHUNKS = E // _CH
    W = _NC * _NS
    K = CHUNKS // W
    REM = CHUNKS - K * W
    RS = (N // _NS) // 8 * 8
    RREM = N - RS * _NS

    @functools.partial(
        pl.kernel,
        out_type=[jax.ShapeDtypeStruct((N,), F32),
                  jax.ShapeDtypeStruct((N,), F32)],
        mesh=_mesh(),
        scratch_types=[
            pltpu.VMEM((_CH,), jnp.int32),
            pltpu.VMEM((_CH,), F32),
            pltpu.VMEM((RS,), F32),
            pltpu.VMEM_SHARED((N,), F32),
        ],
    )
    def deg_kernel(dst_hbm, d0, d1, dst_v, ones_v, init_v, dacc):
        c = lax.axis_index("c")
        t = lax.axis_index("s")
        w = c * _NS + t

        def fill_ones(i, carry):
            ones_v[pl.ds(i * 16, 16)] = jnp.full((16,), 1.0, F32)
            return carry
        lax.fori_loop(0, _CH // 16, fill_ones, 0)
        iv = jnp.where(c == 0, 1.0, 0.0).astype(F32)
        def fill_init(i, carry):
            init_v[pl.ds(i * 16, 16)] = jnp.full((16,), 1.0, F32) * iv
            return carry
        lax.fori_loop(0, RS // 16, fill_init, 0)

        pltpu.sync_copy(init_v, dacc.at[pl.ds(t * RS, RS)])
        @pl.when(t == _NS - 1)
        def _():
            pltpu.sync_copy(init_v.at[pl.ds(0, RREM)],
                            dacc.at[pl.ds(_NS * RS, RREM)])
        plsc.subcore_barrier()

        def body(k, carry):
            b = (w + k * W) * _CH
            pltpu.sync_copy(dst_hbm.at[pl.ds(b, _CH)], dst_v)
            pltpu.sync_copy(ones_v, dacc.at[dst_v], add=True)
            return carry
        lax.fori_loop(0, K, body, 0)
        @pl.when(w < REM)
        def _():
            b = (K * W + w) * _CH
            pltpu.sync_copy(dst_hbm.at[pl.ds(b, _CH)], dst_v)
            pltpu.sync_copy(ones_v, dacc.at[dst_v], add=True)
        plsc.subcore_barrier()

        def drain_to(out):
            pltpu.sync_copy(dacc.at[pl.ds(t * RS, RS)], init_v)
            pltpu.sync_copy(init_v, out.at[pl.ds(t * RS, RS)])
            @pl.when(t == _NS - 1)
            def _():
                pltpu.sync_copy(dacc.at[pl.ds(_NS * RS, RREM)],
                                init_v.at[pl.ds(0, RREM)])
                pltpu.sync_copy(init_v.at[pl.ds(0, RREM)],
                                out.at[pl.ds(_NS * RS, RREM)])

        @pl.when(c == 0)
        def _():
            drain_to(d0)
        @pl.when(c == 1)
        def _():
            drain_to(d1)

    return deg_kernel


def _make_mp(N, E, Fh):
    CHUNKS = E // _CH
    K = CHUNKS // _NS
    REM = CHUNKS - K * _NS
    RS = (N // _NS) // 8 * 8
    RREM = N - RS * _NS
    RB = RS // _CH
    RTAIL = RS - RB * _CH

    @functools.partial(
        pl.kernel,
        out_type=[jax.ShapeDtypeStruct((N, Fh), F32),
                  jax.ShapeDtypeStruct((N, Fh), F32)],
        mesh=_mesh(),
        scratch_types=[
            pltpu.VMEM((_CH,), jnp.int32),
            pltpu.VMEM((_CH,), jnp.int32),
            pltpu.VMEM((_CH, Fh), F32),
            pltpu.VMEM_SHARED((N, Fh), F32),
            pltpu.SemaphoreType.DMA,
        ],
    )
    def mp(m0, m1, src_hbm, dst_hbm, s0, s1, src_v, dst_v, rows_v,
           acc, sem):
        c = lax.axis_index("c")
        t = lax.axis_index("s")
        r0 = t * RS

        def init_from(tab):
            def blk(i, carry):
                off = r0 + i * _CH
                pltpu.sync_copy(tab.at[pl.ds(off, _CH)], rows_v)
                pltpu.sync_copy(rows_v, acc.at[pl.ds(off, _CH)])
                return carry
            lax.fori_loop(0, RB, blk, 0)
            off = r0 + RB * _CH
            pltpu.sync_copy(tab.at[pl.ds(off, RTAIL)],
                            rows_v.at[pl.ds(0, RTAIL)])
            pltpu.sync_copy(rows_v.at[pl.ds(0, RTAIL)],
                            acc.at[pl.ds(off, RTAIL)])
            @pl.when(t == _NS - 1)
            def _():
                pltpu.sync_copy(tab.at[pl.ds(_NS * RS, RREM)],
                                rows_v.at[pl.ds(0, RREM)])
                pltpu.sync_copy(rows_v.at[pl.ds(0, RREM)],
                                acc.at[pl.ds(_NS * RS, RREM)])

        @pl.when(c == 0)
        def _():
            init_from(m0)
        @pl.when(c == 1)
        def _():
            init_from(m1)
        plsc.subcore_barrier()

        def run_edges(tab):
            def step(b):
                pltpu.sync_copy(src_hbm.at[pl.ds(b, _CH)], src_v)
                pltpu.sync_copy(dst_hbm.at[pl.ds(b, _CH)], dst_v)
                pltpu.async_copy(tab.at[src_v], rows_v, sem).wait()
                pltpu.sync_copy(rows_v, acc.at[dst_v], add=True)
            def body(k, carry):
                step((t + k * _NS) * _CH)
                return carry
            lax.fori_loop(0, K, body, 0)
            @pl.when(t < REM)
            def _():
                step((K * _NS + t) * _CH)

        @pl.when(c == 0)
        def _():
            run_edges(m0)
        @pl.when(c == 1)
        def _():
            run_edges(m1)
        plsc.subcore_barrier()

        def drain_to(out):
            def blk(i, carry):
                off = r0 + i * _CH
                pltpu.sync_copy(acc.at[pl.ds(off, _CH)], rows_v)
                pltpu.sync_copy(rows_v, out.at[pl.ds(off, _CH)])
                return carry
            lax.fori_loop(0, RB, blk, 0)
            off = r0 + RB * _CH
            pltpu.sync_copy(acc.at[pl.ds(off, RTAIL)],
                            rows_v.at[pl.ds(0, RTAIL)])
            pltpu.sync_copy(rows_v.at[pl.ds(0, RTAIL)],
                            out.at[pl.ds(off, RTAIL)])
            @pl.when(t == _NS - 1)
            def _():
                pltpu.sync_copy(acc.at[pl.ds(_NS * RS, RREM)],
                                rows_v.at[pl.ds(0, RREM)])
                pltpu.sync_copy(rows_v.at[pl.ds(0, RREM)],
                                out.at[pl.ds(_NS * RS, RREM)])

        @pl.when(c == 0)
        def _():
            drain_to(s0)
        @pl.when(c == 1)
        def _():
            drain_to(s1)

    return mp


def _make_mp2(N, E, F):
    CHUNKS = E // _CH
    HALF = CHUNKS // _NC
    K = HALF // _NS
    REM = HALF - K * _NS
    RS = (N // _NS) // 8 * 8
    RREM = N - RS * _NS
    RB = RS // _CH
    RTAIL = RS - RB * _CH

    @functools.partial(
        pl.kernel,
        out_type=[jax.ShapeDtypeStruct((N, F), F32),
                  jax.ShapeDtypeStruct((N, F), F32)],
        mesh=_mesh(),
        scratch_types=[
            pltpu.VMEM((_CH,), jnp.int32),
            pltpu.VMEM((_CH,), jnp.int32),
            pltpu.VMEM((_CH, F), F32),
            pltpu.VMEM_SHARED((N, F), F32),
            pltpu.SemaphoreType.DMA,
        ],
    )
    def mp2(tab, src_hbm, dst_hbm, s0, s1, src_v, dst_v, rows_v, acc, sem):
        c = lax.axis_index("c")
        t = lax.axis_index("s")
        r0 = t * RS

        def blk_init(i, carry):
            off = r0 + i * _CH
            pltpu.sync_copy(tab.at[pl.ds(off, _CH)], rows_v)
            pltpu.sync_copy(rows_v, acc.at[pl.ds(off, _CH)])
            return carry
        lax.fori_loop(0, RB, blk_init, 0)
        off0 = r0 + RB * _CH
        pltpu.sync_copy(tab.at[pl.ds(off0, RTAIL)],
                        rows_v.at[pl.ds(0, RTAIL)])
        pltpu.sync_copy(rows_v.at[pl.ds(0, RTAIL)],
                        acc.at[pl.ds(off0, RTAIL)])
        @pl.when(t == _NS - 1)
        def _():
            pltpu.sync_copy(tab.at[pl.ds(_NS * RS, RREM)],
                            rows_v.at[pl.ds(0, RREM)])
            pltpu.sync_copy(rows_v.at[pl.ds(0, RREM)],
                            acc.at[pl.ds(_NS * RS, RREM)])
        plsc.subcore_barrier()

        def step(b):
            pltpu.sync_copy(src_hbm.at[pl.ds(b, _CH)], src_v)
            pltpu.sync_copy(dst_hbm.at[pl.ds(b, _CH)], dst_v)
            pltpu.async_copy(tab.at[src_v], rows_v, sem).wait()
            pltpu.sync_copy(rows_v, acc.at[dst_v], add=True)
        def body(k, carry):
            step((c * HALF + t + k * _NS) * _CH)
            return carry
        lax.fori_loop(0, K, body, 0)
        @pl.when(t < REM)
        def _():
            step((c * HALF + K * _NS + t) * _CH)
        plsc.subcore_barrier()

        def drain_to(out):
            def blk(i, carry):
                off = r0 + i * _CH
                pltpu.sync_copy(acc.at[pl.ds(off, _CH)], rows_v)
                pltpu.sync_copy(rows_v, out.at[pl.ds(off, _CH)])
                return carry
            lax.fori_loop(0, RB, blk, 0)
            off = r0 + RB * _CH
            pltpu.sync_copy(acc.at[pl.ds(off, RTAIL)],
                            rows_v.at[pl.ds(0, RTAIL)])
            pltpu.sync_copy(rows_v.at[pl.ds(0, RTAIL)],
                            out.at[pl.ds(off, RTAIL)])
            @pl.when(t == _NS - 1)
            def _():
                pltpu.sync_copy(acc.at[pl.ds(_NS * RS, RREM)],
                                rows_v.at[pl.ds(0, RREM)])
                pltpu.sync_copy(rows_v.at[pl.ds(0, RREM)],
                                out.at[pl.ds(_NS * RS, RREM)])

        @pl.when(c == 0)
        def _():
            drain_to(s0)
        @pl.when(c == 1)
        def _():
            drain_to(s1)

    return mp2


def _tc_a(x, W1, d0r, d1r, N, NB):
    D = x.shape[1]
    F2 = W1.shape[1]
    Fh = F2 // 2

    def body(x_ref, w_ref, d0_ref, d1_ref, m0_ref, m1_ref):
        disq = lax.rsqrt(d0_ref[0, 0] + d1_ref[0, 0])
        p = jnp.dot(x_ref[...], w_ref[...], preferred_element_type=F32)
        p = p * disq[:, None]
        m0_ref[...] = p[:, :Fh]
        m1_ref[...] = p[:, Fh:]

    return pl.pallas_call(
        body,
        grid=(NB,),
        in_specs=[pl.BlockSpec((_BR, D), lambda i: (i, 0)),
                  pl.BlockSpec((D, F2), lambda i: (0, 0)),
                  pl.BlockSpec((1, 1, _BR), lambda i: (i, 0, 0)),
                  pl.BlockSpec((1, 1, _BR), lambda i: (i, 0, 0))],
        out_specs=[pl.BlockSpec((_BR, Fh), lambda i: (i, 0)),
                   pl.BlockSpec((_BR, Fh), lambda i: (i, 0))],
        out_shape=[jax.ShapeDtypeStruct((N, Fh), F32)] * 2,
    )(x, W1, d0r, d1r)


def _tc_b(s0, s1, d0r, d1r, b1r, W2r, N, NB):
    Fh = s0.shape[1]
    H = W2r.shape[3]

    def body(s0_ref, s1_ref, d0_ref, d1_ref, b_ref, w_ref, m_ref):
        disq = lax.rsqrt(d0_ref[0, 0] + d1_ref[0, 0])
        a0 = jnp.maximum(s0_ref[...] * disq[:, None] + b_ref[0, 0][None, :], 0.0)
        a1 = jnp.maximum(s1_ref[...] * disq[:, None] + b_ref[0, 1][None, :], 0.0)
        z = (jnp.dot(a0, w_ref[0, 0], preferred_element_type=F32)
             + jnp.dot(a1, w_ref[0, 1], preferred_element_type=F32))
        m_ref[...] = z * disq[:, None]

    return pl.pallas_call(
        body,
        grid=(NB,),
        in_specs=[pl.BlockSpec((_BR, Fh), lambda i: (i, 0)),
                  pl.BlockSpec((_BR, Fh), lambda i: (i, 0)),
                  pl.BlockSpec((1, 1, _BR), lambda i: (i, 0, 0)),
                  pl.BlockSpec((1, 1, _BR), lambda i: (i, 0, 0)),
                  pl.BlockSpec((1, 2, Fh), lambda i: (0, 0, 0)),
                  pl.BlockSpec((1, 2, Fh, H), lambda i: (0, 0, 0, 0))],
        out_specs=pl.BlockSpec((_BR, H), lambda i: (i, 0)),
        out_shape=jax.ShapeDtypeStruct((N, H), F32),
    )(s0, s1, d0r, d1r, b1r, W2r)


def _tc_c(s0, s1, m2, d0r, d1r, b2r, Wl, blr, N, NB):
    H = s0.shape[1]
    DO = Wl.shape[1]

    def body(s0_ref, s1_ref, m_ref, d0_ref, d1_ref, b_ref, w_ref, bl_ref,
             o_ref):
        disq = lax.rsqrt(d0_ref[0, 0] + d1_ref[0, 0])
        st = s0_ref[...] + s1_ref[...] - m_ref[...]
        h = jnp.maximum(st * disq[:, None] + b_ref[0, 0][None, :], 0.0)
        o_ref[...] = (jnp.dot(h, w_ref[...], preferred_element_type=F32)
                      + bl_ref[0, 0][None, :])

    return pl.pallas_call(
        body,
        grid=(NB,),
        in_specs=[pl.BlockSpec((_BR, H), lambda i: (i, 0)),
                  pl.BlockSpec((_BR, H), lambda i: (i, 0)),
                  pl.BlockSpec((_BR, H), lambda i: (i, 0)),
                  pl.BlockSpec((1, 1, _BR), lambda i: (i, 0, 0)),
                  pl.BlockSpec((1, 1, _BR), lambda i: (i, 0, 0)),
                  pl.BlockSpec((1, 1, H), lambda i: (0, 0, 0)),
                  pl.BlockSpec((H, DO), lambda i: (0, 0)),
                  pl.BlockSpec((1, 1, DO), lambda i: (0, 0, 0))],
        out_specs=pl.BlockSpec((_BR, DO), lambda i: (i, 0)),
        out_shape=jax.ShapeDtypeStruct((N, DO), F32),
    )(s0, s1, m2, d0r, d1r, b2r, Wl, blr)


def kernel(x, edge_attrs, edge_index, W1, b1, W2, b2, Wl, bl):
    del edge_attrs
    N, D = x.shape
    E = edge_index.shape[1]
    F2 = W1.shape[1]
    H = W2.shape[1]
    DO = Wl.shape[1]
    NB = N // _BR

    src = edge_index[0]
    dst = edge_index[1]

    d0, d1 = _make_deg(N, E)(dst)
    d0r = d0.reshape(NB, 1, _BR)
    d1r = d1.reshape(NB, 1, _BR)

    m1_0, m1_1 = _tc_a(x, W1, d0r, d1r, N, NB)
    s1_0, s1_1 = _make_mp(N, E, F2 // 2)(m1_0, m1_1, src, dst)
    m2 = _tc_b(s1_0, s1_1, d0r, d1r, b1.reshape(1, 2, F2 // 2),
               W2.reshape(1, 2, F2 // 2, H), N, NB)
    s2_0, s2_1 = _make_mp2(N, E, H)(m2, src, dst)
    return _tc_c(s2_0, s2_1, m2, d0r, d1r, b2.reshape(1, 1, H), Wl,
                 bl.reshape(1, 1, DO), N, NB)


# speedup vs baseline: 19.3445x; 1.4991x over previous
import functools

import jax
import jax.numpy as jnp
from jax import lax
from jax.experimental import pallas as pl
from jax.experimental.pallas import tpu as pltpu
from jax.experimental.pallas import tpu_sc as plsc

F32 = jnp.float32
_NS = 16
_NC = 2
_CH = 128
_BR = 400


def _mesh():
    return plsc.VectorSubcoreMesh(core_axis_name="c", subcore_axis_name="s")


def _make_deg(N, E):
    CHUNKS = E // _CH
    W = _NC * _NS
    K = CHUNKS // W
    REM = CHUNKS - K * W
    RS = (N // _NS) // 8 * 8
    RREM = N - RS * _NS

    @functools.partial(
        pl.kernel,
        out_type=[jax.ShapeDtypeStruct((N,), F32),
                  jax.ShapeDtypeStruct((N,), F32)],
        mesh=_mesh(),
        scratch_types=[
            pltpu.VMEM((_CH,), jnp.int32),
            pltpu.VMEM((_CH,), F32),
            pltpu.VMEM((RS,), F32),
            pltpu.VMEM_SHARED((N,), F32),
        ],
    )
    def deg_kernel(dst_hbm, d0, d1, dst_v, ones_v, init_v, dacc):
        c = lax.axis_index("c")
        t = lax.axis_index("s")
        w = c * _NS + t

        def fill_ones(i, carry):
            ones_v[pl.ds(i * 16, 16)] = jnp.full((16,), 1.0, F32)
            return carry
        lax.fori_loop(0, _CH // 16, fill_ones, 0)
        iv = jnp.where(c == 0, 1.0, 0.0).astype(F32)
        def fill_init(i, carry):
            init_v[pl.ds(i * 16, 16)] = jnp.full((16,), 1.0, F32) * iv
            return carry
        lax.fori_loop(0, RS // 16, fill_init, 0)

        pltpu.sync_copy(init_v, dacc.at[pl.ds(t * RS, RS)])
        @pl.when(t == _NS - 1)
        def _():
            pltpu.sync_copy(init_v.at[pl.ds(0, RREM)],
                            dacc.at[pl.ds(_NS * RS, RREM)])
        plsc.subcore_barrier()

        def body(k, carry):
            b = (w + k * W) * _CH
            pltpu.sync_copy(dst_hbm.at[pl.ds(b, _CH)], dst_v)
            pltpu.sync_copy(ones_v, dacc.at[dst_v], add=True)
            return carry
        lax.fori_loop(0, K, body, 0)
        @pl.when(w < REM)
        def _():
            b = (K * W + w) * _CH
            pltpu.sync_copy(dst_hbm.at[pl.ds(b, _CH)], dst_v)
            pltpu.sync_copy(ones_v, dacc.at[dst_v], add=True)
        plsc.subcore_barrier()

        def drain_to(out):
            pltpu.sync_copy(dacc.at[pl.ds(t * RS, RS)], init_v)
            pltpu.sync_copy(init_v, out.at[pl.ds(t * RS, RS)])
            @pl.when(t == _NS - 1)
            def _():
                pltpu.sync_copy(dacc.at[pl.ds(_NS * RS, RREM)],
                                init_v.at[pl.ds(0, RREM)])
                pltpu.sync_copy(init_v.at[pl.ds(0, RREM)],
                                out.at[pl.ds(_NS * RS, RREM)])

        @pl.when(c == 0)
        def _():
            drain_to(d0)
        @pl.when(c == 1)
        def _():
            drain_to(d1)

    return deg_kernel


def _make_mp(N, E, Fh):
    CHUNKS = E // _CH
    K = CHUNKS // _NS
    REM = CHUNKS - K * _NS
    PAIRS = K // 2
    KTAIL = K - 2 * PAIRS
    RS = (N // _NS) // 8 * 8
    RREM = N - RS * _NS
    RB = RS // _CH
    RTAIL = RS - RB * _CH

    @functools.partial(
        pl.kernel,
        out_type=[jax.ShapeDtypeStruct((N, Fh), F32),
                  jax.ShapeDtypeStruct((N, Fh), F32)],
        mesh=_mesh(),
        scratch_types=[
            pltpu.VMEM((_CH,), jnp.int32),
            pltpu.VMEM((_CH,), jnp.int32),
            pltpu.VMEM((_CH,), jnp.int32),
            pltpu.VMEM((_CH,), jnp.int32),
            pltpu.VMEM((_CH, Fh), F32),
            pltpu.VMEM((_CH, Fh), F32),
            pltpu.VMEM_SHARED((N, Fh), F32),
            pltpu.SemaphoreType.DMA,
            pltpu.SemaphoreType.DMA,
        ],
    )
    def mp(m0, m1, src_hbm, dst_hbm, s0, s1, src_v, dst_v, src_w, dst_w,
           rows_v, rows_w, acc, sem_v, sem_w):
        c = lax.axis_index("c")
        t = lax.axis_index("s")
        r0 = t * RS

        def init_from(tab):
            def blk(i, carry):
                off = r0 + i * _CH
                pltpu.sync_copy(tab.at[pl.ds(off, _CH)], rows_v)
                pltpu.sync_copy(rows_v, acc.at[pl.ds(off, _CH)])
                return carry
            lax.fori_loop(0, RB, blk, 0)
            off = r0 + RB * _CH
            pltpu.sync_copy(tab.at[pl.ds(off, RTAIL)],
                            rows_v.at[pl.ds(0, RTAIL)])
            pltpu.sync_copy(rows_v.at[pl.ds(0, RTAIL)],
                            acc.at[pl.ds(off, RTAIL)])
            @pl.when(t == _NS - 1)
            def _():
                pltpu.sync_copy(tab.at[pl.ds(_NS * RS, RREM)],
                                rows_v.at[pl.ds(0, RREM)])
                pltpu.sync_copy(rows_v.at[pl.ds(0, RREM)],
                                acc.at[pl.ds(_NS * RS, RREM)])

        @pl.when(c == 0)
        def _():
            init_from(m0)
        @pl.when(c == 1)
        def _():
            init_from(m1)
        plsc.subcore_barrier()

        def run_edges(tab):
            bufs = ((src_v, dst_v, rows_v, sem_v),
                    (src_w, dst_w, rows_w, sem_w))
            def start(ck, buf):
                s, d, r, sm = buf
                pltpu.sync_copy(src_hbm.at[pl.ds(ck * _CH, _CH)], s)
                pltpu.sync_copy(dst_hbm.at[pl.ds(ck * _CH, _CH)], d)
                pltpu.async_copy(tab.at[s], r, sm)
            def finish(buf):
                s, d, r, sm = buf
                pltpu.make_async_copy(tab.at[s], r, sm).wait()
                pltpu.sync_copy(r, acc.at[d], add=True)
            def serial(ck):
                start(ck, bufs[0])
                finish(bufs[0])
            if PAIRS > 0:
                start(t, bufs[0])
                def body(i, carry):
                    start(t + (2 * i + 1) * _NS, bufs[1])
                    finish(bufs[0])
                    @pl.when(i + 1 < PAIRS)
                    def _():
                        start(t + (2 * i + 2) * _NS, bufs[0])
                    finish(bufs[1])
                    return carry
                lax.fori_loop(0, PAIRS, body, 0)
            if KTAIL:
                serial(t + (K - 1) * _NS)
            @pl.when(t < REM)
            def _():
                serial(K * _NS + t)

        @pl.when(c == 0)
        def _():
            run_edges(m0)
        @pl.when(c == 1)
        def _():
            run_edges(m1)
        plsc.subcore_barrier()

        def drain_to(out):
            def blk(i, carry):
                off = r0 + i * _CH
                pltpu.sync_copy(acc.at[pl.ds(off, _CH)], rows_v)
                pltpu.sync_copy(rows_v, out.at[pl.ds(off, _CH)])
                return carry
            lax.fori_loop(0, RB, blk, 0)
            off = r0 + RB * _CH
            pltpu.sync_copy(acc.at[pl.ds(off, RTAIL)],
                            rows_v.at[pl.ds(0, RTAIL)])
            pltpu.sync_copy(rows_v.at[pl.ds(0, RTAIL)],
                            out.at[pl.ds(off, RTAIL)])
            @pl.when(t == _NS - 1)
            def _():
                pltpu.sync_copy(acc.at[pl.ds(_NS * RS, RREM)],
                                rows_v.at[pl.ds(0, RREM)])
                pltpu.sync_copy(rows_v.at[pl.ds(0, RREM)],
                                out.at[pl.ds(_NS * RS, RREM)])

        @pl.when(c == 0)
        def _():
            drain_to(s0)
        @pl.when(c == 1)
        def _():
            drain_to(s1)

    return mp


def _make_mp2(N, E, F):
    CHUNKS = E // _CH
    HALF = CHUNKS // _NC
    K = HALF // _NS
    REM = HALF - K * _NS
    PAIRS = K // 2
    KTAIL = K - 2 * PAIRS
    RS = (N // _NS) // 8 * 8
    RREM = N - RS * _NS
    RB = RS // _CH
    RTAIL = RS - RB * _CH

    @functools.partial(
        pl.kernel,
        out_type=[jax.ShapeDtypeStruct((N, F), F32),
                  jax.ShapeDtypeStruct((N, F), F32)],
        mesh=_mesh(),
        scratch_types=[
            pltpu.VMEM((_CH,), jnp.int32),
            pltpu.VMEM((_CH,), jnp.int32),
            pltpu.VMEM((_CH,), jnp.int32),
            pltpu.VMEM((_CH,), jnp.int32),
            pltpu.VMEM((_CH, F), F32),
            pltpu.VMEM((_CH, F), F32),
            pltpu.VMEM_SHARED((N, F), F32),
            pltpu.SemaphoreType.DMA,
            pltpu.SemaphoreType.DMA,
        ],
    )
    def mp2(tab, src_hbm, dst_hbm, s0, s1, src_v, dst_v, src_w, dst_w,
            rows_v, rows_w, acc, sem_v, sem_w):
        c = lax.axis_index("c")
        t = lax.axis_index("s")
        r0 = t * RS

        def blk_init(i, carry):
            off = r0 + i * _CH
            pltpu.sync_copy(tab.at[pl.ds(off, _CH)], rows_v)
            pltpu.sync_copy(rows_v, acc.at[pl.ds(off, _CH)])
            return carry
        lax.fori_loop(0, RB, blk_init, 0)
        off0 = r0 + RB * _CH
        pltpu.sync_copy(tab.at[pl.ds(off0, RTAIL)],
                        rows_v.at[pl.ds(0, RTAIL)])
        pltpu.sync_copy(rows_v.at[pl.ds(0, RTAIL)],
                        acc.at[pl.ds(off0, RTAIL)])
        @pl.when(t == _NS - 1)
        def _():
            pltpu.sync_copy(tab.at[pl.ds(_NS * RS, RREM)],
                            rows_v.at[pl.ds(0, RREM)])
            pltpu.sync_copy(rows_v.at[pl.ds(0, RREM)],
                            acc.at[pl.ds(_NS * RS, RREM)])
        plsc.subcore_barrier()

        bufs = ((src_v, dst_v, rows_v, sem_v),
                (src_w, dst_w, rows_w, sem_w))
        def start(ck, buf):
            s, d, r, sm = buf
            pltpu.sync_copy(src_hbm.at[pl.ds(ck * _CH, _CH)], s)
            pltpu.sync_copy(dst_hbm.at[pl.ds(ck * _CH, _CH)], d)
            pltpu.async_copy(tab.at[s], r, sm)
        def finish(buf):
            s, d, r, sm = buf
            pltpu.make_async_copy(tab.at[s], r, sm).wait()
            pltpu.sync_copy(r, acc.at[d], add=True)
        def serial(ck):
            start(ck, bufs[0])
            finish(bufs[0])
        c0 = c * HALF + t
        if PAIRS > 0:
            start(c0, bufs[0])
            def body(i, carry):
                start(c0 + (2 * i + 1) * _NS, bufs[1])
                finish(bufs[0])
                @pl.when(i + 1 < PAIRS)
                def _():
                    start(c0 + (2 * i + 2) * _NS, bufs[0])
                finish(bufs[1])
                return carry
            lax.fori_loop(0, PAIRS, body, 0)
        if KTAIL:
            serial(c0 + (K - 1) * _NS)
        @pl.when(t < REM)
        def _():
            serial(c * HALF + K * _NS + t)
        plsc.subcore_barrier()

        def drain_to(out):
            def blk(i, carry):
                off = r0 + i * _CH
                pltpu.sync_copy(acc.at[pl.ds(off, _CH)], rows_v)
                pltpu.sync_copy(rows_v, out.at[pl.ds(off, _CH)])
                return carry
            lax.fori_loop(0, RB, blk, 0)
            off = r0 + RB * _CH
            pltpu.sync_copy(acc.at[pl.ds(off, RTAIL)],
                            rows_v.at[pl.ds(0, RTAIL)])
            pltpu.sync_copy(rows_v.at[pl.ds(0, RTAIL)],
                            out.at[pl.ds(off, RTAIL)])
            @pl.when(t == _NS - 1)
            def _():
                pltpu.sync_copy(acc.at[pl.ds(_NS * RS, RREM)],
                                rows_v.at[pl.ds(0, RREM)])
                pltpu.sync_copy(rows_v.at[pl.ds(0, RREM)],
                                out.at[pl.ds(_NS * RS, RREM)])

        @pl.when(c == 0)
        def _():
            drain_to(s0)
        @pl.when(c == 1)
        def _():
            drain_to(s1)

    return mp2


def _tc_a(x, W1, d0r, d1r, N, NB):
    D = x.shape[1]
    F2 = W1.shape[1]
    Fh = F2 // 2

    def body(x_ref, w_ref, d0_ref, d1_ref, m0_ref, m1_ref):
        disq = lax.rsqrt(d0_ref[0, 0] + d1_ref[0, 0])
        p = jnp.dot(x_ref[...], w_ref[...], preferred_element_type=F32)
        p = p * disq[:, None]
        m0_ref[...] = p[:, :Fh]
        m1_ref[...] = p[:, Fh:]

    return pl.pallas_call(
        body,
        grid=(NB,),
        in_specs=[pl.BlockSpec((_BR, D), lambda i: (i, 0)),
                  pl.BlockSpec((D, F2), lambda i: (0, 0)),
                  pl.BlockSpec((1, 1, _BR), lambda i: (i, 0, 0)),
                  pl.BlockSpec((1, 1, _BR), lambda i: (i, 0, 0))],
        out_specs=[pl.BlockSpec((_BR, Fh), lambda i: (i, 0)),
                   pl.BlockSpec((_BR, Fh), lambda i: (i, 0))],
        out_shape=[jax.ShapeDtypeStruct((N, Fh), F32)] * 2,
    )(x, W1, d0r, d1r)


def _tc_b(s0, s1, d0r, d1r, b1r, W2r, N, NB):
    Fh = s0.shape[1]
    H = W2r.shape[3]

    def body(s0_ref, s1_ref, d0_ref, d1_ref, b_ref, w_ref, m_ref):
        disq = lax.rsqrt(d0_ref[0, 0] + d1_ref[0, 0])
        a0 = jnp.maximum(s0_ref[...] * disq[:, None] + b_ref[0, 0][None, :], 0.0)
        a1 = jnp.maximum(s1_ref[...] * disq[:, None] + b_ref[0, 1][None, :], 0.0)
        z = (jnp.dot(a0, w_ref[0, 0], preferred_element_type=F32)
             + jnp.dot(a1, w_ref[0, 1], preferred_element_type=F32))
        m_ref[...] = z * disq[:, None]

    return pl.pallas_call(
        body,
        grid=(NB,),
        in_specs=[pl.BlockSpec((_BR, Fh), lambda i: (i, 0)),
                  pl.BlockSpec((_BR, Fh), lambda i: (i, 0)),
                  pl.BlockSpec((1, 1, _BR), lambda i: (i, 0, 0)),
                  pl.BlockSpec((1, 1, _BR), lambda i: (i, 0, 0)),
                  pl.BlockSpec((1, 2, Fh), lambda i: (0, 0, 0)),
                  pl.BlockSpec((1, 2, Fh, H), lambda i: (0, 0, 0, 0))],
        out_specs=pl.BlockSpec((_BR, H), lambda i: (i, 0)),
        out_shape=jax.ShapeDtypeStruct((N, H), F32),
    )(s0, s1, d0r, d1r, b1r, W2r)


def _tc_c(s0, s1, m2, d0r, d1r, b2r, Wl, blr, N, NB):
    H = s0.shape[1]
    DO = Wl.shape[1]

    def body(s0_ref, s1_ref, m_ref, d0_ref, d1_ref, b_ref, w_ref, bl_ref,
             o_ref):
        disq = lax.rsqrt(d0_ref[0, 0] + d1_ref[0, 0])
        st = s0_ref[...] + s1_ref[...] - m_ref[...]
        h = jnp.maximum(st * disq[:, None] + b_ref[0, 0][None, :], 0.0)
        o_ref[...] = (jnp.dot(h, w_ref[...], preferred_element_type=F32)
                      + bl_ref[0, 0][None, :])

    return pl.pallas_call(
        body,
        grid=(NB,),
        in_specs=[pl.BlockSpec((_BR, H), lambda i: (i, 0)),
                  pl.BlockSpec((_BR, H), lambda i: (i, 0)),
                  pl.BlockSpec((_BR, H), lambda i: (i, 0)),
                  pl.BlockSpec((1, 1, _BR), lambda i: (i, 0, 0)),
                  pl.BlockSpec((1, 1, _BR), lambda i: (i, 0, 0)),
                  pl.BlockSpec((1, 1, H), lambda i: (0, 0, 0)),
                  pl.BlockSpec((H, DO), lambda i: (0, 0)),
                  pl.BlockSpec((1, 1, DO), lambda i: (0, 0, 0))],
        out_specs=pl.BlockSpec((_BR, DO), lambda i: (i, 0)),
        out_shape=jax.ShapeDtypeStruct((N, DO), F32),
    )(s0, s1, m2, d0r, d1r, b2r, Wl, blr)


def kernel(x, edge_attrs, edge_index, W1, b1, W2, b2, Wl, bl):
    del edge_attrs
    N, D = x.shape
    E = edge_index.shape[1]
    F2 = W1.shape[1]
    H = W2.shape[1]
    DO = Wl.shape[1]
    NB = N // _BR

    src = edge_index[0]
    dst = edge_index[1]

    d0, d1 = _make_deg(N, E)(dst)
    d0r = d0.reshape(NB, 1, _BR)
    d1r = d1.reshape(NB, 1, _BR)

    m1_0, m1_1 = _tc_a(x, W1, d0r, d1r, N, NB)
    s1_0, s1_1 = _make_mp(N, E, F2 // 2)(m1_0, m1_1, src, dst)
    m2 = _tc_b(s1_0, s1_1, d0r, d1r, b1.reshape(1, 2, F2 // 2),
               W2.reshape(1, 2, F2 // 2, H), N, NB)
    s2_0, s2_1 = _make_mp2(N, E, H)(m2, src, dst)
    return _tc_c(s2_0, s2_1, m2, d0r, d1r, b2.reshape(1, 1, H), Wl,
                 bl.reshape(1, 1, DO), N, NB)


# speedup vs baseline: 24.8709x; 1.2857x over previous
import functools

import jax
import jax.numpy as jnp
from jax import lax
from jax.experimental import pallas as pl
from jax.experimental.pallas import tpu as pltpu
from jax.experimental.pallas import tpu_sc as plsc

F32 = jnp.float32
_NS = 16
_NC = 2
_CH = 128
_BR = 400


def _mesh():
    return plsc.VectorSubcoreMesh(core_axis_name="c", subcore_axis_name="s")


def _make_deg(N, E):
    CHUNKS = E // _CH
    W = _NC * _NS
    K = CHUNKS // W
    REM = CHUNKS - K * W
    RS = (N // _NS) // 8 * 8
    RREM = N - RS * _NS

    @functools.partial(
        pl.kernel,
        out_type=[jax.ShapeDtypeStruct((N,), F32),
                  jax.ShapeDtypeStruct((N,), F32)],
        mesh=_mesh(),
        scratch_types=[
            pltpu.VMEM((_CH,), jnp.int32),
            pltpu.VMEM((_CH,), F32),
            pltpu.VMEM((RS,), F32),
            pltpu.VMEM_SHARED((N,), F32),
        ],
    )
    def deg_kernel(dst_hbm, d0, d1, dst_v, ones_v, init_v, dacc):
        c = lax.axis_index("c")
        t = lax.axis_index("s")
        w = c * _NS + t

        def fill_ones(i, carry):
            ones_v[pl.ds(i * 16, 16)] = jnp.full((16,), 1.0, F32)
            return carry
        lax.fori_loop(0, _CH // 16, fill_ones, 0)
        iv = jnp.where(c == 0, 1.0, 0.0).astype(F32)
        def fill_init(i, carry):
            init_v[pl.ds(i * 16, 16)] = jnp.full((16,), 1.0, F32) * iv
            return carry
        lax.fori_loop(0, RS // 16, fill_init, 0)

        pltpu.sync_copy(init_v, dacc.at[pl.ds(t * RS, RS)])
        @pl.when(t == _NS - 1)
        def _():
            pltpu.sync_copy(init_v.at[pl.ds(0, RREM)],
                            dacc.at[pl.ds(_NS * RS, RREM)])
        plsc.subcore_barrier()

        def body(k, carry):
            b = (w + k * W) * _CH
            pltpu.sync_copy(dst_hbm.at[pl.ds(b, _CH)], dst_v)
            pltpu.sync_copy(ones_v, dacc.at[dst_v], add=True)
            return carry
        lax.fori_loop(0, K, body, 0)
        @pl.when(w < REM)
        def _():
            b = (K * W + w) * _CH
            pltpu.sync_copy(dst_hbm.at[pl.ds(b, _CH)], dst_v)
            pltpu.sync_copy(ones_v, dacc.at[dst_v], add=True)
        plsc.subcore_barrier()

        def drain_to(out):
            pltpu.sync_copy(dacc.at[pl.ds(t * RS, RS)], init_v)
            pltpu.sync_copy(init_v, out.at[pl.ds(t * RS, RS)])
            @pl.when(t == _NS - 1)
            def _():
                pltpu.sync_copy(dacc.at[pl.ds(_NS * RS, RREM)],
                                init_v.at[pl.ds(0, RREM)])
                pltpu.sync_copy(init_v.at[pl.ds(0, RREM)],
                                out.at[pl.ds(_NS * RS, RREM)])

        @pl.when(c == 0)
        def _():
            drain_to(d0)
        @pl.when(c == 1)
        def _():
            drain_to(d1)

    return deg_kernel


def _edge_pipeline(src_hbm, dst_hbm, tab, acc, idx_slots, row_slots, c0, K,
                   rem_pred):
    def ck(k):
        return c0 + k * _NS

    def start_idx(k, m):
        s, d, smi = idx_slots[m]
        off = ck(k) * _CH
        pltpu.async_copy(src_hbm.at[pl.ds(off, _CH)], s, smi)
        pltpu.async_copy(dst_hbm.at[pl.ds(off, _CH)], d, smi)

    def wait_idx(m):
        s, d, smi = idx_slots[m]
        pltpu.make_async_copy(src_hbm.at[pl.ds(0, _CH)], s, smi).wait()
        pltpu.make_async_copy(dst_hbm.at[pl.ds(0, _CH)], d, smi).wait()

    def issue_gather(p, m):
        r, smg, _ = row_slots[p]
        pltpu.async_copy(tab.at[idx_slots[m][0]], r, smg)

    def wait_gather(p, m):
        r, smg, _ = row_slots[p]
        pltpu.make_async_copy(tab.at[idx_slots[m][0]], r, smg).wait()

    def issue_scatter(p, m):
        r, _, sms = row_slots[p]
        pltpu.async_copy(r, acc.at[idx_slots[m][1]], sms, add=True)

    def wait_scatter(p, m):
        r, _, sms = row_slots[p]
        pltpu.make_async_copy(r, acc.at[idx_slots[m][1]], sms).wait()

    def serial(k):
        s, d, smi = idx_slots[0]
        r = row_slots[0][0]
        off = ck(k) * _CH
        pltpu.sync_copy(src_hbm.at[pl.ds(off, _CH)], s)
        pltpu.sync_copy(dst_hbm.at[pl.ds(off, _CH)], d)
        issue_gather(0, 0)
        wait_gather(0, 0)
        issue_scatter(0, 0)
        wait_scatter(0, 0)

    def tail():
        @pl.when(rem_pred)
        def _():
            serial(K)

    J = K // 6
    if J == 0:
        for k in range(K):
            serial(k)
        tail()
        return

    start_idx(0, 0)
    start_idx(1, 1)
    wait_idx(0)
    issue_gather(0, 0)

    def body(j, carry):
        for b in range(6):
            k = 6 * j + b
            if b < 5:
                wait_idx((b + 1) % 3)
            else:
                @pl.when(j < J - 1)
                def _():
                    wait_idx(0)
            if b > 0:
                wait_scatter((b + 1) % 2, (b + 2) % 3)
            else:
                @pl.when(j > 0)
                def _():
                    wait_scatter(1, 2)
            if b < 4:
                start_idx(k + 2, (b + 2) % 3)
            else:
                @pl.when(j < J - 1)
                def _():
                    start_idx(k + 2, (b + 2) % 3)
            if b < 5:
                issue_gather((b + 1) % 2, (b + 1) % 3)
            else:
                @pl.when(j < J - 1)
                def _():
                    issue_gather(0, 0)
            wait_gather(b % 2, b % 3)
            issue_scatter(b % 2, b % 3)
        return carry
    lax.fori_loop(0, J, body, 0)
    wait_scatter((K - 1) % 2, (K - 1) % 3)
    for k in range(6 * J, K):
        serial(k)
    tail()


def _make_mp(N, E, Fh):
    CHUNKS = E // _CH
    K = CHUNKS // _NS
    REM = CHUNKS - K * _NS
    PAIRS = K // 2
    KTAIL = K - 2 * PAIRS
    RS = (N // _NS) // 8 * 8
    RREM = N - RS * _NS
    RB = RS // _CH
    RTAIL = RS - RB * _CH

    @functools.partial(
        pl.kernel,
        out_type=[jax.ShapeDtypeStruct((N, Fh), F32),
                  jax.ShapeDtypeStruct((N, Fh), F32)],
        mesh=_mesh(),
        scratch_types=[
            pltpu.VMEM((_CH,), jnp.int32),
            pltpu.VMEM((_CH,), jnp.int32),
            pltpu.VMEM((_CH,), jnp.int32),
            pltpu.VMEM((_CH,), jnp.int32),
            pltpu.VMEM((_CH,), jnp.int32),
            pltpu.VMEM((_CH,), jnp.int32),
            pltpu.VMEM((_CH, Fh), F32),
            pltpu.VMEM((_CH, Fh), F32),
            pltpu.VMEM_SHARED((N, Fh), F32),
            pltpu.SemaphoreType.DMA,
            pltpu.SemaphoreType.DMA,
            pltpu.SemaphoreType.DMA,
            pltpu.SemaphoreType.DMA,
            pltpu.SemaphoreType.DMA,
            pltpu.SemaphoreType.DMA,
            pltpu.SemaphoreType.DMA,
        ],
    )
    def mp(m0, m1, src_hbm, dst_hbm, s0, s1, si0, di0, si1, di1, si2, di2,
           rows_v, rows_w, acc, smi0, smi1, smi2, smg0, smg1, sms0, sms1):
        c = lax.axis_index("c")
        t = lax.axis_index("s")
        r0 = t * RS
        idx_slots = ((si0, di0, smi0), (si1, di1, smi1), (si2, di2, smi2))
        row_slots = ((rows_v, smg0, sms0), (rows_w, smg1, sms1))

        def init_from(tab):
            def blk(i, carry):
                off = r0 + i * _CH
                pltpu.sync_copy(tab.at[pl.ds(off, _CH)], rows_v)
                pltpu.sync_copy(rows_v, acc.at[pl.ds(off, _CH)])
                return carry
            lax.fori_loop(0, RB, blk, 0)
            off = r0 + RB * _CH
            pltpu.sync_copy(tab.at[pl.ds(off, RTAIL)],
                            rows_v.at[pl.ds(0, RTAIL)])
            pltpu.sync_copy(rows_v.at[pl.ds(0, RTAIL)],
                            acc.at[pl.ds(off, RTAIL)])
            @pl.when(t == _NS - 1)
            def _():
                pltpu.sync_copy(tab.at[pl.ds(_NS * RS, RREM)],
                                rows_v.at[pl.ds(0, RREM)])
                pltpu.sync_copy(rows_v.at[pl.ds(0, RREM)],
                                acc.at[pl.ds(_NS * RS, RREM)])

        @pl.when(c == 0)
        def _():
            init_from(m0)
        @pl.when(c == 1)
        def _():
            init_from(m1)
        plsc.subcore_barrier()

        def run_edges(tab):
            _edge_pipeline(src_hbm, dst_hbm, tab, acc, idx_slots, row_slots,
                           t, K, t < REM)

        @pl.when(c == 0)
        def _():
            run_edges(m0)
        @pl.when(c == 1)
        def _():
            run_edges(m1)
        plsc.subcore_barrier()

        def drain_to(out):
            def blk(i, carry):
                off = r0 + i * _CH
                pltpu.sync_copy(acc.at[pl.ds(off, _CH)], rows_v)
                pltpu.sync_copy(rows_v, out.at[pl.ds(off, _CH)])
                return carry
            lax.fori_loop(0, RB, blk, 0)
            off = r0 + RB * _CH
            pltpu.sync_copy(acc.at[pl.ds(off, RTAIL)],
                            rows_v.at[pl.ds(0, RTAIL)])
            pltpu.sync_copy(rows_v.at[pl.ds(0, RTAIL)],
                            out.at[pl.ds(off, RTAIL)])
            @pl.when(t == _NS - 1)
            def _():
                pltpu.sync_copy(acc.at[pl.ds(_NS * RS, RREM)],
                                rows_v.at[pl.ds(0, RREM)])
                pltpu.sync_copy(rows_v.at[pl.ds(0, RREM)],
                                out.at[pl.ds(_NS * RS, RREM)])

        @pl.when(c == 0)
        def _():
            drain_to(s0)
        @pl.when(c == 1)
        def _():
            drain_to(s1)

    return mp


def _make_mp2(N, E, F):
    CHUNKS = E // _CH
    HALF = CHUNKS // _NC
    K = HALF // _NS
    REM = HALF - K * _NS
    PAIRS = K // 2
    KTAIL = K - 2 * PAIRS
    RS = (N // _NS) // 8 * 8
    RREM = N - RS * _NS
    RB = RS // _CH
    RTAIL = RS - RB * _CH

    @functools.partial(
        pl.kernel,
        out_type=[jax.ShapeDtypeStruct((N, F), F32),
                  jax.ShapeDtypeStruct((N, F), F32)],
        mesh=_mesh(),
        scratch_types=[
            pltpu.VMEM((_CH,), jnp.int32),
            pltpu.VMEM((_CH,), jnp.int32),
            pltpu.VMEM((_CH,), jnp.int32),
            pltpu.VMEM((_CH,), jnp.int32),
            pltpu.VMEM((_CH,), jnp.int32),
            pltpu.VMEM((_CH,), jnp.int32),
            pltpu.VMEM((_CH, F), F32),
            pltpu.VMEM((_CH, F), F32),
            pltpu.VMEM_SHARED((N, F), F32),
            pltpu.SemaphoreType.DMA,
            pltpu.SemaphoreType.DMA,
            pltpu.SemaphoreType.DMA,
            pltpu.SemaphoreType.DMA,
            pltpu.SemaphoreType.DMA,
            pltpu.SemaphoreType.DMA,
            pltpu.SemaphoreType.DMA,
        ],
    )
    def mp2(tab, src_hbm, dst_hbm, s0, s1, si0, di0, si1, di1, si2, di2,
            rows_v, rows_w, acc, smi0, smi1, smi2, smg0, smg1, sms0, sms1):
        c = lax.axis_index("c")
        t = lax.axis_index("s")
        r0 = t * RS
        idx_slots = ((si0, di0, smi0), (si1, di1, smi1), (si2, di2, smi2))
        row_slots = ((rows_v, smg0, sms0), (rows_w, smg1, sms1))

        def blk_init(i, carry):
            off = r0 + i * _CH
            pltpu.sync_copy(tab.at[pl.ds(off, _CH)], rows_v)
            pltpu.sync_copy(rows_v, acc.at[pl.ds(off, _CH)])
            return carry
        lax.fori_loop(0, RB, blk_init, 0)
        off0 = r0 + RB * _CH
        pltpu.sync_copy(tab.at[pl.ds(off0, RTAIL)],
                        rows_v.at[pl.ds(0, RTAIL)])
        pltpu.sync_copy(rows_v.at[pl.ds(0, RTAIL)],
                        acc.at[pl.ds(off0, RTAIL)])
        @pl.when(t == _NS - 1)
        def _():
            pltpu.sync_copy(tab.at[pl.ds(_NS * RS, RREM)],
                            rows_v.at[pl.ds(0, RREM)])
            pltpu.sync_copy(rows_v.at[pl.ds(0, RREM)],
                            acc.at[pl.ds(_NS * RS, RREM)])
        plsc.subcore_barrier()

        _edge_pipeline(src_hbm, dst_hbm, tab, acc, idx_slots, row_slots,
                       c * HALF + t, K, t < REM)
        plsc.subcore_barrier()

        def drain_to(out):
            def blk(i, carry):
                off = r0 + i * _CH
                pltpu.sync_copy(acc.at[pl.ds(off, _CH)], rows_v)
                pltpu.sync_copy(rows_v, out.at[pl.ds(off, _CH)])
                return carry
            lax.fori_loop(0, RB, blk, 0)
            off = r0 + RB * _CH
            pltpu.sync_copy(acc.at[pl.ds(off, RTAIL)],
                            rows_v.at[pl.ds(0, RTAIL)])
            pltpu.sync_copy(rows_v.at[pl.ds(0, RTAIL)],
                            out.at[pl.ds(off, RTAIL)])
            @pl.when(t == _NS - 1)
            def _():
                pltpu.sync_copy(acc.at[pl.ds(_NS * RS, RREM)],
                                rows_v.at[pl.ds(0, RREM)])
                pltpu.sync_copy(rows_v.at[pl.ds(0, RREM)],
                                out.at[pl.ds(_NS * RS, RREM)])

        @pl.when(c == 0)
        def _():
            drain_to(s0)
        @pl.when(c == 1)
        def _():
            drain_to(s1)

    return mp2


def _tc_a(x, W1, d0r, d1r, N, NB):
    D = x.shape[1]
    F2 = W1.shape[1]
    Fh = F2 // 2

    def body(x_ref, w_ref, d0_ref, d1_ref, m0_ref, m1_ref):
        disq = lax.rsqrt(d0_ref[0, 0] + d1_ref[0, 0])
        p = jnp.dot(x_ref[...], w_ref[...], preferred_element_type=F32)
        p = p * disq[:, None]
        m0_ref[...] = p[:, :Fh]
        m1_ref[...] = p[:, Fh:]

    return pl.pallas_call(
        body,
        grid=(NB,),
        in_specs=[pl.BlockSpec((_BR, D), lambda i: (i, 0)),
                  pl.BlockSpec((D, F2), lambda i: (0, 0)),
                  pl.BlockSpec((1, 1, _BR), lambda i: (i, 0, 0)),
                  pl.BlockSpec((1, 1, _BR), lambda i: (i, 0, 0))],
        out_specs=[pl.BlockSpec((_BR, Fh), lambda i: (i, 0)),
                   pl.BlockSpec((_BR, Fh), lambda i: (i, 0))],
        out_shape=[jax.ShapeDtypeStruct((N, Fh), F32)] * 2,
    )(x, W1, d0r, d1r)


def _tc_b(s0, s1, d0r, d1r, b1r, W2r, N, NB):
    Fh = s0.shape[1]
    H = W2r.shape[3]

    def body(s0_ref, s1_ref, d0_ref, d1_ref, b_ref, w_ref, m_ref):
        disq = lax.rsqrt(d0_ref[0, 0] + d1_ref[0, 0])
        a0 = jnp.maximum(s0_ref[...] * disq[:, None] + b_ref[0, 0][None, :], 0.0)
        a1 = jnp.maximum(s1_ref[...] * disq[:, None] + b_ref[0, 1][None, :], 0.0)
        z = (jnp.dot(a0, w_ref[0, 0], preferred_element_type=F32)
             + jnp.dot(a1, w_ref[0, 1], preferred_element_type=F32))
        m_ref[...] = z * disq[:, None]

    return pl.pallas_call(
        body,
        grid=(NB,),
        in_specs=[pl.BlockSpec((_BR, Fh), lambda i: (i, 0)),
                  pl.BlockSpec((_BR, Fh), lambda i: (i, 0)),
                  pl.BlockSpec((1, 1, _BR), lambda i: (i, 0, 0)),
                  pl.BlockSpec((1, 1, _BR), lambda i: (i, 0, 0)),
                  pl.BlockSpec((1, 2, Fh), lambda i: (0, 0, 0)),
                  pl.BlockSpec((1, 2, Fh, H), lambda i: (0, 0, 0, 0))],
        out_specs=pl.BlockSpec((_BR, H), lambda i: (i, 0)),
        out_shape=jax.ShapeDtypeStruct((N, H), F32),
    )(s0, s1, d0r, d1r, b1r, W2r)


def _tc_c(s0, s1, m2, d0r, d1r, b2r, Wl, blr, N, NB):
    H = s0.shape[1]
    DO = Wl.shape[1]

    def body(s0_ref, s1_ref, m_ref, d0_ref, d1_ref, b_ref, w_ref, bl_ref,
             o_ref):
        disq = lax.rsqrt(d0_ref[0, 0] + d1_ref[0, 0])
        st = s0_ref[...] + s1_ref[...] - m_ref[...]
        h = jnp.maximum(st * disq[:, None] + b_ref[0, 0][None, :], 0.0)
        o_ref[...] = (jnp.dot(h, w_ref[...], preferred_element_type=F32)
                      + bl_ref[0, 0][None, :])

    return pl.pallas_call(
        body,
        grid=(NB,),
        in_specs=[pl.BlockSpec((_BR, H), lambda i: (i, 0)),
                  pl.BlockSpec((_BR, H), lambda i: (i, 0)),
                  pl.BlockSpec((_BR, H), lambda i: (i, 0)),
                  pl.BlockSpec((1, 1, _BR), lambda i: (i, 0, 0)),
                  pl.BlockSpec((1, 1, _BR), lambda i: (i, 0, 0)),
                  pl.BlockSpec((1, 1, H), lambda i: (0, 0, 0)),
                  pl.BlockSpec((H, DO), lambda i: (0, 0)),
                  pl.BlockSpec((1, 1, DO), lambda i: (0, 0, 0))],
        out_specs=pl.BlockSpec((_BR, DO), lambda i: (i, 0)),
        out_shape=jax.ShapeDtypeStruct((N, DO), F32),
    )(s0, s1, m2, d0r, d1r, b2r, Wl, blr)


def kernel(x, edge_attrs, edge_index, W1, b1, W2, b2, Wl, bl):
    del edge_attrs
    N, D = x.shape
    E = edge_index.shape[1]
    F2 = W1.shape[1]
    H = W2.shape[1]
    DO = Wl.shape[1]
    NB = N // _BR

    src = edge_index[0]
    dst = edge_index[1]

    d0, d1 = _make_deg(N, E)(dst)
    d0r = d0.reshape(NB, 1, _BR)
    d1r = d1.reshape(NB, 1, _BR)

    m1_0, m1_1 = _tc_a(x, W1, d0r, d1r, N, NB)
    s1_0, s1_1 = _make_mp(N, E, F2 // 2)(m1_0, m1_1, src, dst)
    m2 = _tc_b(s1_0, s1_1, d0r, d1r, b1.reshape(1, 2, F2 // 2),
               W2.reshape(1, 2, F2 // 2, H), N, NB)
    s2_0, s2_1 = _make_mp2(N, E, H)(m2, src, dst)
    return _tc_c(s2_0, s2_1, m2, d0r, d1r, b2.reshape(1, 1, H), Wl,
                 bl.reshape(1, 1, DO), N, NB)


# speedup vs baseline: 25.2787x; 1.0164x over previous
import functools

import jax
import jax.numpy as jnp
from jax import lax
from jax.experimental import pallas as pl
from jax.experimental.pallas import tpu as pltpu
from jax.experimental.pallas import tpu_sc as plsc

F32 = jnp.float32
_NS = 16
_NC = 2
_CH = 128
_BR = 400


def _mesh():
    return plsc.VectorSubcoreMesh(core_axis_name="c", subcore_axis_name="s")


def _make_deg(N, E):
    CHUNKS = E // _CH
    W = _NC * _NS
    K = CHUNKS // W
    REM = CHUNKS - K * W
    RS = (N // _NS) // 8 * 8
    RREM = N - RS * _NS

    @functools.partial(
        pl.kernel,
        out_type=[jax.ShapeDtypeStruct((N,), F32),
                  jax.ShapeDtypeStruct((N,), F32)],
        mesh=_mesh(),
        scratch_types=[
            pltpu.VMEM((_CH,), jnp.int32),
            pltpu.VMEM((_CH,), jnp.int32),
            pltpu.VMEM((_CH,), jnp.int32),
            pltpu.VMEM((_CH,), F32),
            pltpu.VMEM((RS,), F32),
            pltpu.VMEM_SHARED((N,), F32),
            pltpu.SemaphoreType.DMA,
            pltpu.SemaphoreType.DMA,
            pltpu.SemaphoreType.DMA,
            pltpu.SemaphoreType.DMA,
        ],
    )
    def deg_kernel(dst_hbm, d0, d1, dst_v, dst_w, dst_x, ones_v, init_v,
                   dacc, smi0, smi1, smi2, sms):
        c = lax.axis_index("c")
        t = lax.axis_index("s")
        w = c * _NS + t
        dsl = (dst_v, dst_w, dst_x)
        smi = (smi0, smi1, smi2)

        def fill_ones(i, carry):
            ones_v[pl.ds(i * 16, 16)] = jnp.full((16,), 1.0, F32)
            return carry
        lax.fori_loop(0, _CH // 16, fill_ones, 0)
        iv = jnp.where(c == 0, 1.0, 0.0).astype(F32)
        def fill_init(i, carry):
            init_v[pl.ds(i * 16, 16)] = jnp.full((16,), 1.0, F32) * iv
            return carry
        lax.fori_loop(0, RS // 16, fill_init, 0)

        pltpu.sync_copy(init_v, dacc.at[pl.ds(t * RS, RS)])
        @pl.when(t == _NS - 1)
        def _():
            pltpu.sync_copy(init_v.at[pl.ds(0, RREM)],
                            dacc.at[pl.ds(_NS * RS, RREM)])
        plsc.subcore_barrier()

        def start_idx(k, m):
            pltpu.async_copy(dst_hbm.at[pl.ds((w + k * W) * _CH, _CH)],
                             dsl[m], smi[m])
        def wait_idx(m):
            pltpu.make_async_copy(dst_hbm.at[pl.ds(0, _CH)], dsl[m],
                                  smi[m]).wait()
        def issue_scatter(m):
            pltpu.async_copy(ones_v, dacc.at[dsl[m]], sms, add=True)
        def wait_scatter(m):
            pltpu.make_async_copy(ones_v, dacc.at[dsl[m]], sms).wait()

        J = K // 6
        start_idx(0, 0)
        start_idx(1, 1)
        wait_idx(0)
        def body(j, carry):
            for b in range(6):
                k = 6 * j + b
                if b < 5:
                    wait_idx((b + 1) % 3)
                else:
                    @pl.when(j < J - 1)
                    def _():
                        wait_idx(0)
                if b > 0:
                    wait_scatter((b + 2) % 3)
                else:
                    @pl.when(j > 0)
                    def _():
                        wait_scatter(2)
                if b < 4:
                    start_idx(k + 2, (b + 2) % 3)
                else:
                    @pl.when(j < J - 1)
                    def _():
                        start_idx(k + 2, (b + 2) % 3)
                issue_scatter(b % 3)
            return carry
        lax.fori_loop(0, J, body, 0)
        wait_scatter((6 * J - 1) % 3)
        for k in range(6 * J, K):
            pltpu.sync_copy(dst_hbm.at[pl.ds((w + k * W) * _CH, _CH)], dst_v)
            issue_scatter(0)
            wait_scatter(0)
        @pl.when(w < REM)
        def _():
            pltpu.sync_copy(dst_hbm.at[pl.ds((K * W + w) * _CH, _CH)], dst_v)
            issue_scatter(0)
            wait_scatter(0)
        plsc.subcore_barrier()

        def drain_to(out):
            pltpu.sync_copy(dacc.at[pl.ds(t * RS, RS)], init_v)
            pltpu.sync_copy(init_v, out.at[pl.ds(t * RS, RS)])
            @pl.when(t == _NS - 1)
            def _():
                pltpu.sync_copy(dacc.at[pl.ds(_NS * RS, RREM)],
                                init_v.at[pl.ds(0, RREM)])
                pltpu.sync_copy(init_v.at[pl.ds(0, RREM)],
                                out.at[pl.ds(_NS * RS, RREM)])

        @pl.when(c == 0)
        def _():
            drain_to(d0)
        @pl.when(c == 1)
        def _():
            drain_to(d1)

    return deg_kernel


def _edge_pipeline(src_hbm, dst_hbm, tab, acc, idx_slots, row_slots, c0, K,
                   rem_pred):
    def ck(k):
        return c0 + k * _NS

    def start_idx(k, m):
        s, d, smi = idx_slots[m]
        off = ck(k) * _CH
        pltpu.async_copy(src_hbm.at[pl.ds(off, _CH)], s, smi)
        pltpu.async_copy(dst_hbm.at[pl.ds(off, _CH)], d, smi)

    def wait_idx(m):
        s, d, smi = idx_slots[m]
        pltpu.make_async_copy(src_hbm.at[pl.ds(0, _CH)], s, smi).wait()
        pltpu.make_async_copy(dst_hbm.at[pl.ds(0, _CH)], d, smi).wait()

    def issue_gather(p, m):
        r, smg, _ = row_slots[p]
        pltpu.async_copy(tab.at[idx_slots[m][0]], r, smg)

    def wait_gather(p, m):
        r, smg, _ = row_slots[p]
        pltpu.make_async_copy(tab.at[idx_slots[m][0]], r, smg).wait()

    def issue_scatter(p, m):
        r, _, sms = row_slots[p]
        pltpu.async_copy(r, acc.at[idx_slots[m][1]], sms, add=True)

    def wait_scatter(p, m):
        r, _, sms = row_slots[p]
        pltpu.make_async_copy(r, acc.at[idx_slots[m][1]], sms).wait()

    def serial(k):
        s, d, smi = idx_slots[0]
        r = row_slots[0][0]
        off = ck(k) * _CH
        pltpu.sync_copy(src_hbm.at[pl.ds(off, _CH)], s)
        pltpu.sync_copy(dst_hbm.at[pl.ds(off, _CH)], d)
        issue_gather(0, 0)
        wait_gather(0, 0)
        issue_scatter(0, 0)
        wait_scatter(0, 0)

    def tail():
        @pl.when(rem_pred)
        def _():
            serial(K)

    J = K // 6
    if J == 0:
        for k in range(K):
            serial(k)
        tail()
        return

    start_idx(0, 0)
    start_idx(1, 1)
    wait_idx(0)
    issue_gather(0, 0)

    def body(j, carry):
        for b in range(6):
            k = 6 * j + b
            if b < 5:
                wait_idx((b + 1) % 3)
            else:
                @pl.when(j < J - 1)
                def _():
                    wait_idx(0)
            if b > 0:
                wait_scatter((b + 1) % 2, (b + 2) % 3)
            else:
                @pl.when(j > 0)
                def _():
                    wait_scatter(1, 2)
            if b < 4:
                start_idx(k + 2, (b + 2) % 3)
            else:
                @pl.when(j < J - 1)
                def _():
                    start_idx(k + 2, (b + 2) % 3)
            if b < 5:
                issue_gather((b + 1) % 2, (b + 1) % 3)
            else:
                @pl.when(j < J - 1)
                def _():
                    issue_gather(0, 0)
            wait_gather(b % 2, b % 3)
            issue_scatter(b % 2, b % 3)
        return carry
    lax.fori_loop(0, J, body, 0)
    wait_scatter((K - 1) % 2, (K - 1) % 3)
    for k in range(6 * J, K):
        serial(k)
    tail()


def _make_mp(N, E, Fh):
    CHUNKS = E // _CH
    K = CHUNKS // _NS
    REM = CHUNKS - K * _NS
    PAIRS = K // 2
    KTAIL = K - 2 * PAIRS
    RS = (N // _NS) // 8 * 8
    RREM = N - RS * _NS
    RB = RS // _CH
    RTAIL = RS - RB * _CH

    @functools.partial(
        pl.kernel,
        out_type=[jax.ShapeDtypeStruct((N, Fh), F32),
                  jax.ShapeDtypeStruct((N, Fh), F32)],
        mesh=_mesh(),
        scratch_types=[
            pltpu.VMEM((_CH,), jnp.int32),
            pltpu.VMEM((_CH,), jnp.int32),
            pltpu.VMEM((_CH,), jnp.int32),
            pltpu.VMEM((_CH,), jnp.int32),
            pltpu.VMEM((_CH,), jnp.int32),
            pltpu.VMEM((_CH,), jnp.int32),
            pltpu.VMEM((_CH, Fh), F32),
            pltpu.VMEM((_CH, Fh), F32),
            pltpu.VMEM_SHARED((N, Fh), F32),
            pltpu.SemaphoreType.DMA,
            pltpu.SemaphoreType.DMA,
            pltpu.SemaphoreType.DMA,
            pltpu.SemaphoreType.DMA,
            pltpu.SemaphoreType.DMA,
            pltpu.SemaphoreType.DMA,
            pltpu.SemaphoreType.DMA,
        ],
    )
    def mp(m0, m1, src_hbm, dst_hbm, s0, s1, si0, di0, si1, di1, si2, di2,
           rows_v, rows_w, acc, smi0, smi1, smi2, smg0, smg1, sms0, sms1):
        c = lax.axis_index("c")
        t = lax.axis_index("s")
        r0 = t * RS
        idx_slots = ((si0, di0, smi0), (si1, di1, smi1), (si2, di2, smi2))
        row_slots = ((rows_v, smg0, sms0), (rows_w, smg1, sms1))

        def init_from(tab):
            def blk(i, carry):
                off = r0 + i * _CH
                pltpu.sync_copy(tab.at[pl.ds(off, _CH)], rows_v)
                pltpu.sync_copy(rows_v, acc.at[pl.ds(off, _CH)])
                return carry
            lax.fori_loop(0, RB, blk, 0)
            off = r0 + RB * _CH
            pltpu.sync_copy(tab.at[pl.ds(off, RTAIL)],
                            rows_v.at[pl.ds(0, RTAIL)])
            pltpu.sync_copy(rows_v.at[pl.ds(0, RTAIL)],
                            acc.at[pl.ds(off, RTAIL)])
            @pl.when(t == _NS - 1)
            def _():
                pltpu.sync_copy(tab.at[pl.ds(_NS * RS, RREM)],
                                rows_v.at[pl.ds(0, RREM)])
                pltpu.sync_copy(rows_v.at[pl.ds(0, RREM)],
                                acc.at[pl.ds(_NS * RS, RREM)])

        @pl.when(c == 0)
        def _():
            init_from(m0)
        @pl.when(c == 1)
        def _():
            init_from(m1)
        plsc.subcore_barrier()

        def run_edges(tab):
            _edge_pipeline(src_hbm, dst_hbm, tab, acc, idx_slots, row_slots,
                           t, K, t < REM)

        @pl.when(c == 0)
        def _():
            run_edges(m0)
        @pl.when(c == 1)
        def _():
            run_edges(m1)
        plsc.subcore_barrier()

        def drain_to(out):
            def blk(i, carry):
                off = r0 + i * _CH
                pltpu.sync_copy(acc.at[pl.ds(off, _CH)], rows_v)
                pltpu.sync_copy(rows_v, out.at[pl.ds(off, _CH)])
                return carry
            lax.fori_loop(0, RB, blk, 0)
            off = r0 + RB * _CH
            pltpu.sync_copy(acc.at[pl.ds(off, RTAIL)],
                            rows_v.at[pl.ds(0, RTAIL)])
            pltpu.sync_copy(rows_v.at[pl.ds(0, RTAIL)],
                            out.at[pl.ds(off, RTAIL)])
            @pl.when(t == _NS - 1)
            def _():
                pltpu.sync_copy(acc.at[pl.ds(_NS * RS, RREM)],
                                rows_v.at[pl.ds(0, RREM)])
                pltpu.sync_copy(rows_v.at[pl.ds(0, RREM)],
                                out.at[pl.ds(_NS * RS, RREM)])

        @pl.when(c == 0)
        def _():
            drain_to(s0)
        @pl.when(c == 1)
        def _():
            drain_to(s1)

    return mp


def _make_mp2(N, E, F):
    CHUNKS = E // _CH
    HALF = CHUNKS // _NC
    K = HALF // _NS
    REM = HALF - K * _NS
    PAIRS = K // 2
    KTAIL = K - 2 * PAIRS
    RS = (N // _NS) // 8 * 8
    RREM = N - RS * _NS
    RB = RS // _CH
    RTAIL = RS - RB * _CH

    @functools.partial(
        pl.kernel,
        out_type=[jax.ShapeDtypeStruct((N, F), F32),
                  jax.ShapeDtypeStruct((N, F), F32)],
        mesh=_mesh(),
        scratch_types=[
            pltpu.VMEM((_CH,), jnp.int32),
            pltpu.VMEM((_CH,), jnp.int32),
            pltpu.VMEM((_CH,), jnp.int32),
            pltpu.VMEM((_CH,), jnp.int32),
            pltpu.VMEM((_CH,), jnp.int32),
            pltpu.VMEM((_CH,), jnp.int32),
            pltpu.VMEM((_CH, F), F32),
            pltpu.VMEM((_CH, F), F32),
            pltpu.VMEM_SHARED((N, F), F32),
            pltpu.SemaphoreType.DMA,
            pltpu.SemaphoreType.DMA,
            pltpu.SemaphoreType.DMA,
            pltpu.SemaphoreType.DMA,
            pltpu.SemaphoreType.DMA,
            pltpu.SemaphoreType.DMA,
            pltpu.SemaphoreType.DMA,
        ],
    )
    def mp2(tab, src_hbm, dst_hbm, s0, s1, si0, di0, si1, di1, si2, di2,
            rows_v, rows_w, acc, smi0, smi1, smi2, smg0, smg1, sms0, sms1):
        c = lax.axis_index("c")
        t = lax.axis_index("s")
        r0 = t * RS
        idx_slots = ((si0, di0, smi0), (si1, di1, smi1), (si2, di2, smi2))
        row_slots = ((rows_v, smg0, sms0), (rows_w, smg1, sms1))

        def blk_init(i, carry):
            off = r0 + i * _CH
            pltpu.sync_copy(tab.at[pl.ds(off, _CH)], rows_v)
            pltpu.sync_copy(rows_v, acc.at[pl.ds(off, _CH)])
            return carry
        lax.fori_loop(0, RB, blk_init, 0)
        off0 = r0 + RB * _CH
        pltpu.sync_copy(tab.at[pl.ds(off0, RTAIL)],
                        rows_v.at[pl.ds(0, RTAIL)])
        pltpu.sync_copy(rows_v.at[pl.ds(0, RTAIL)],
                        acc.at[pl.ds(off0, RTAIL)])
        @pl.when(t == _NS - 1)
        def _():
            pltpu.sync_copy(tab.at[pl.ds(_NS * RS, RREM)],
                            rows_v.at[pl.ds(0, RREM)])
            pltpu.sync_copy(rows_v.at[pl.ds(0, RREM)],
                            acc.at[pl.ds(_NS * RS, RREM)])
        plsc.subcore_barrier()

        _edge_pipeline(src_hbm, dst_hbm, tab, acc, idx_slots, row_slots,
                       c * HALF + t, K, t < REM)
        plsc.subcore_barrier()

        def drain_to(out):
            def blk(i, carry):
                off = r0 + i * _CH
                pltpu.sync_copy(acc.at[pl.ds(off, _CH)], rows_v)
                pltpu.sync_copy(rows_v, out.at[pl.ds(off, _CH)])
                return carry
            lax.fori_loop(0, RB, blk, 0)
            off = r0 + RB * _CH
            pltpu.sync_copy(acc.at[pl.ds(off, RTAIL)],
                            rows_v.at[pl.ds(0, RTAIL)])
            pltpu.sync_copy(rows_v.at[pl.ds(0, RTAIL)],
                            out.at[pl.ds(off, RTAIL)])
            @pl.when(t == _NS - 1)
            def _():
                pltpu.sync_copy(acc.at[pl.ds(_NS * RS, RREM)],
                                rows_v.at[pl.ds(0, RREM)])
                pltpu.sync_copy(rows_v.at[pl.ds(0, RREM)],
                                out.at[pl.ds(_NS * RS, RREM)])

        @pl.when(c == 0)
        def _():
            drain_to(s0)
        @pl.when(c == 1)
        def _():
            drain_to(s1)

    return mp2


def _tc_a(x, W1, d0r, d1r, N, NB):
    D = x.shape[1]
    F2 = W1.shape[1]
    Fh = F2 // 2

    def body(x_ref, w_ref, d0_ref, d1_ref, m0_ref, m1_ref):
        disq = lax.rsqrt(d0_ref[0, 0] + d1_ref[0, 0])
        p = jnp.dot(x_ref[...], w_ref[...], preferred_element_type=F32)
        p = p * disq[:, None]
        m0_ref[...] = p[:, :Fh]
        m1_ref[...] = p[:, Fh:]

    return pl.pallas_call(
        body,
        grid=(NB,),
        in_specs=[pl.BlockSpec((_BR, D), lambda i: (i, 0)),
                  pl.BlockSpec((D, F2), lambda i: (0, 0)),
                  pl.BlockSpec((1, 1, _BR), lambda i: (i, 0, 0)),
                  pl.BlockSpec((1, 1, _BR), lambda i: (i, 0, 0))],
        out_specs=[pl.BlockSpec((_BR, Fh), lambda i: (i, 0)),
                   pl.BlockSpec((_BR, Fh), lambda i: (i, 0))],
        out_shape=[jax.ShapeDtypeStruct((N, Fh), F32)] * 2,
    )(x, W1, d0r, d1r)


def _tc_b(s0, s1, d0r, d1r, b1r, W2r, N, NB):
    Fh = s0.shape[1]
    H = W2r.shape[3]

    def body(s0_ref, s1_ref, d0_ref, d1_ref, b_ref, w_ref, m_ref):
        disq = lax.rsqrt(d0_ref[0, 0] + d1_ref[0, 0])
        a0 = jnp.maximum(s0_ref[...] * disq[:, None] + b_ref[0, 0][None, :], 0.0)
        a1 = jnp.maximum(s1_ref[...] * disq[:, None] + b_ref[0, 1][None, :], 0.0)
        z = (jnp.dot(a0, w_ref[0, 0], preferred_element_type=F32)
             + jnp.dot(a1, w_ref[0, 1], preferred_element_type=F32))
        m_ref[...] = z * disq[:, None]

    return pl.pallas_call(
        body,
        grid=(NB,),
        in_specs=[pl.BlockSpec((_BR, Fh), lambda i: (i, 0)),
                  pl.BlockSpec((_BR, Fh), lambda i: (i, 0)),
                  pl.BlockSpec((1, 1, _BR), lambda i: (i, 0, 0)),
                  pl.BlockSpec((1, 1, _BR), lambda i: (i, 0, 0)),
                  pl.BlockSpec((1, 2, Fh), lambda i: (0, 0, 0)),
                  pl.BlockSpec((1, 2, Fh, H), lambda i: (0, 0, 0, 0))],
        out_specs=pl.BlockSpec((_BR, H), lambda i: (i, 0)),
        out_shape=jax.ShapeDtypeStruct((N, H), F32),
    )(s0, s1, d0r, d1r, b1r, W2r)


def _tc_c(s0, s1, m2, d0r, d1r, b2r, Wl, blr, N, NB):
    H = s0.shape[1]
    DO = Wl.shape[1]

    def body(s0_ref, s1_ref, m_ref, d0_ref, d1_ref, b_ref, w_ref, bl_ref,
             o_ref):
        disq = lax.rsqrt(d0_ref[0, 0] + d1_ref[0, 0])
        st = s0_ref[...] + s1_ref[...] - m_ref[...]
        h = jnp.maximum(st * disq[:, None] + b_ref[0, 0][None, :], 0.0)
        o_ref[...] = (jnp.dot(h, w_ref[...], preferred_element_type=F32)
                      + bl_ref[0, 0][None, :])

    return pl.pallas_call(
        body,
        grid=(NB,),
        in_specs=[pl.BlockSpec((_BR, H), lambda i: (i, 0)),
                  pl.BlockSpec((_BR, H), lambda i: (i, 0)),
                  pl.BlockSpec((_BR, H), lambda i: (i, 0)),
                  pl.BlockSpec((1, 1, _BR), lambda i: (i, 0, 0)),
                  pl.BlockSpec((1, 1, _BR), lambda i: (i, 0, 0)),
                  pl.BlockSpec((1, 1, H), lambda i: (0, 0, 0)),
                  pl.BlockSpec((H, DO), lambda i: (0, 0)),
                  pl.BlockSpec((1, 1, DO), lambda i: (0, 0, 0))],
        out_specs=pl.BlockSpec((_BR, DO), lambda i: (i, 0)),
        out_shape=jax.ShapeDtypeStruct((N, DO), F32),
    )(s0, s1, m2, d0r, d1r, b2r, Wl, blr)


def kernel(x, edge_attrs, edge_index, W1, b1, W2, b2, Wl, bl):
    del edge_attrs
    N, D = x.shape
    E = edge_index.shape[1]
    F2 = W1.shape[1]
    H = W2.shape[1]
    DO = Wl.shape[1]
    NB = N // _BR

    src = edge_index[0]
    dst = edge_index[1]

    d0, d1 = _make_deg(N, E)(dst)
    d0r = d0.reshape(NB, 1, _BR)
    d1r = d1.reshape(NB, 1, _BR)

    m1_0, m1_1 = _tc_a(x, W1, d0r, d1r, N, NB)
    s1_0, s1_1 = _make_mp(N, E, F2 // 2)(m1_0, m1_1, src, dst)
    m2 = _tc_b(s1_0, s1_1, d0r, d1r, b1.reshape(1, 2, F2 // 2),
               W2.reshape(1, 2, F2 // 2, H), N, NB)
    s2_0, s2_1 = _make_mp2(N, E, H)(m2, src, dst)
    return _tc_c(s2_0, s2_1, m2, d0r, d1r, b2.reshape(1, 1, H), Wl,
                 bl.reshape(1, 1, DO), N, NB)
